# baseline (speedup 1.0000x reference)
import jax
import jax.numpy as jnp
from jax.experimental import pallas as pl
from jax.experimental.pallas import tpu as pltpu


def _shift_rows(a, s):
    if s == 0:
        return a
    z = jnp.zeros((1, a.shape[1]), a.dtype)
    if s == 1:
        return jnp.concatenate([a[1:], z], axis=0)
    return jnp.concatenate([z, a[:-1]], axis=0)


def _shift_cols(a, s):
    if s == 0:
        return a
    z = jnp.zeros((a.shape[0], 1), a.dtype)
    if s == 1:
        return jnp.concatenate([a[:, 1:], z], axis=1)
    return jnp.concatenate([z, a[:, :-1]], axis=1)


def _tree_sum(terms):
    while len(terms) > 1:
        nxt = [terms[i] + terms[i + 1] for i in range(0, len(terms) - 1, 2)]
        if len(terms) % 2:
            nxt.append(terms[-1])
        terms = nxt
    return terms[0]


def _conv3x3_vpu_kernel(x_ref, w_ref, b_ref, o_ref, r_ref):
    _, cin, H, W = x_ref.shape
    cout = o_ref.shape[1]

    for ci in range(cin):
        base = x_ref[0, ci]
        for dh in range(3):
            r_ref[ci * 3 + dh] = _shift_rows(base, dh - 1).astype(jnp.bfloat16)

    for co in range(cout):
        acc = jnp.full((H, W), b_ref[co], jnp.float32)
        for dw in range(3):
            terms = []
            for k in range(cin * 3):
                ci, dh = divmod(k, 3)
                w_splat = pltpu.bitcast(
                    jnp.full((H // 2, W), w_ref[co, ci, dh, dw], jnp.uint32),
                    jnp.bfloat16)
                terms.append(r_ref[k] * w_splat)
            q = _tree_sum(terms).astype(jnp.float32)
            acc = acc + _shift_cols(q, dw - 1)
        o_ref[0, co] = acc


@jax.jit
def _conv_impl(x_nchw, weight_oihw, bias):
    N, Cin, H, W = x_nchw.shape
    Cout = weight_oihw.shape[0]
    w_u16 = jax.lax.bitcast_convert_type(
        weight_oihw.astype(jnp.bfloat16), jnp.uint16).astype(jnp.uint32)
    w_dup = w_u16 | (w_u16 << 16)
    return pl.pallas_call(
        _conv3x3_vpu_kernel,
        out_shape=jax.ShapeDtypeStruct((N, Cout, H, W), jnp.float32),
        grid=(N,),
        in_specs=[
            pl.BlockSpec((1, Cin, H, W), lambda n: (n, 0, 0, 0)),
            pl.BlockSpec(memory_space=pltpu.SMEM),
            pl.BlockSpec(memory_space=pltpu.SMEM),
        ],
        out_specs=pl.BlockSpec((1, Cout, H, W), lambda n: (n, 0, 0, 0)),
        scratch_shapes=[pltpu.VMEM((Cin * 3, H, W), jnp.bfloat16)],
        compiler_params=pltpu.CompilerParams(
            dimension_semantics=("parallel",),
            vmem_limit_bytes=32 * 1024 * 1024,
        ),
    )(x_nchw, w_dup, bias).astype(x_nchw.dtype)


def kernel(x_nchw, weight_oihw, bias):
    return _conv_impl(x_nchw, weight_oihw, bias)


# speedup vs baseline: 1.4045x; 1.4045x over previous
import jax
import jax.numpy as jnp
from jax.experimental import pallas as pl
from jax.experimental.pallas import tpu as pltpu


def _shift_rows(a, s):
    if s == 0:
        return a
    z = jnp.zeros((1, a.shape[1]), a.dtype)
    if s == 1:
        return jnp.concatenate([a[1:], z], axis=0)
    return jnp.concatenate([z, a[:-1]], axis=0)


def _conv3x3_mxu_kernel(x_ref, m_ref, b_ref, o_ref, lhs_ref):
    imgs, cin, H, W = x_ref.shape
    cout = o_ref.shape[1]

    for i in range(imgs):
        for ci in range(cin):
            base = x_ref[i, ci]
            for dh in range(3):
                k = ci * 3 + dh
                lhs_ref[i * H:(i + 1) * H, k * W:(k + 1) * W] = (
                    _shift_rows(base, dh - 1))

    wide = jnp.dot(lhs_ref[...], m_ref[...],
                   preferred_element_type=jnp.float32)

    for i in range(imgs):
        for co in range(cout):
            o_ref[i, co] = (wide[i * H:(i + 1) * H, co * W:(co + 1) * W]
                            + b_ref[co])


def _build_w_toeplitz(weight_oihw, W):
    p = jnp.arange(W)
    dw = jnp.arange(3)
    sel = (p[None, :, None] ==
           (p[None, None, :] + dw[:, None, None] - 1)).astype(jnp.float32)
    cout, cin = weight_oihw.shape[:2]
    m = jnp.einsum("dpq,ockd->ckpoq", sel, weight_oihw)
    return m.reshape(cin * 3 * W, cout * W)


@jax.jit
def _conv_impl(x_nchw, weight_oihw, bias):
    N, Cin, H, W = x_nchw.shape
    Cout = weight_oihw.shape[0]
    IMGS = 2
    m = _build_w_toeplitz(weight_oihw, W)
    return pl.pallas_call(
        _conv3x3_mxu_kernel,
        out_shape=jax.ShapeDtypeStruct((N, Cout, H, W), jnp.float32),
        grid=(N // IMGS,),
        in_specs=[
            pl.BlockSpec((IMGS, Cin, H, W), lambda n: (n, 0, 0, 0)),
            pl.BlockSpec((Cin * 3 * W, Cout * W), lambda n: (0, 0)),
            pl.BlockSpec(memory_space=pltpu.SMEM),
        ],
        out_specs=pl.BlockSpec((IMGS, Cout, H, W), lambda n: (n, 0, 0, 0)),
        scratch_shapes=[pltpu.VMEM((IMGS * H, Cin * 3 * W), jnp.float32)],
        compiler_params=pltpu.CompilerParams(
            dimension_semantics=("parallel",),
            vmem_limit_bytes=48 * 1024 * 1024,
        ),
    )(x_nchw, m, bias).astype(x_nchw.dtype)


def kernel(x_nchw, weight_oihw, bias):
    return _conv_impl(x_nchw, weight_oihw, bias)
